# Initial kernel scaffold; baseline (speedup 1.0000x reference)
import jax
import jax.numpy as jnp
from jax.experimental import pallas as pl


def kernel(q_sub, q_rel, hidden, edges, n_node, time_stamp, rel_table, pe_rel, pe_abs, gate_W_w, gate_W_b, gate_ht_w, gate_ht_b, fuse_w1, fuse_b1, fuse_w2, fuse_b2, Ws_w, Wr_w, Wqr_w, Wqr_b, walpha_w, Wh_w):
    raise NotImplementedError("write your pallas kernel here")


# speedup vs baseline: 2.5790x; 2.5790x over previous
import functools

import jax
import jax.numpy as jnp
from jax import lax
from jax.experimental import pallas as pl
from jax.experimental.pallas import tpu as pltpu
from jax.experimental.pallas import tpu_sc as plsc

D = 128
DP = 32
DA = 64
NCORE = 2
NSUB = 16
NW = NCORE * NSUB
C = 80


def _lrelu(t):
    return jnp.where(t >= 0, t, 0.01 * t)


def _make_gather(NE, NN, NR, NQ, EPW, NCHUNK):
    mesh = plsc.VectorSubcoreMesh(core_axis_name="c", subcore_axis_name="s")
    f32 = jnp.float32

    @functools.partial(
        pl.kernel,
        out_type=(
            jax.ShapeDtypeStruct((NE, D), f32),
            jax.ShapeDtypeStruct((NE, D), f32),
            jax.ShapeDtypeStruct((NE, D), f32),
            jax.ShapeDtypeStruct((NE, DP), f32),
            jax.ShapeDtypeStruct((NE, DP), f32),
        ),
        mesh=mesh,
        scratch_types=[
            pltpu.VMEM((NQ,), jnp.int32),
            pltpu.VMEM((EPW,), jnp.int32),
            pltpu.VMEM((EPW,), jnp.int32),
            pltpu.VMEM((EPW,), jnp.int32),
            pltpu.VMEM((EPW,), jnp.int32),
            pltpu.VMEM((EPW,), jnp.int32),
            pltpu.VMEM((EPW,), jnp.int32),
            pltpu.VMEM((C, D), f32),
            pltpu.VMEM((C, D), f32),
            pltpu.VMEM((C, D), f32),
            pltpu.VMEM((C, DP), f32),
            pltpu.VMEM((C, DP), f32),
            pltpu.SemaphoreType.DMA,
        ],
        compiler_params=pltpu.CompilerParams(needs_layout_passes=False,
                                             use_tc_tiling_on_sc=False),
    )
    def gather_k(rel_hbm, hid_hbm, per_hbm, pea_hbm, qrel_hbm,
                 ec2, ec6, etab, esub, erel,
                 o_rel, o_hs, o_qr, o_per, o_pea,
                 qrelv, c2v, c6v, tabv, subv, relv, qidxv,
                 b_rel, b_hs, b_qr, b_per, b_pea, sem):
        wid = lax.axis_index("s") * NCORE + lax.axis_index("c")
        base = wid * EPW
        pltpu.sync_copy(qrel_hbm, qrelv)
        pltpu.sync_copy(ec2.at[pl.ds(base, EPW)], c2v)
        pltpu.sync_copy(ec6.at[pl.ds(base, EPW)], c6v)
        pltpu.sync_copy(etab.at[pl.ds(base, EPW)], tabv)
        pltpu.sync_copy(esub.at[pl.ds(base, EPW)], subv)
        pltpu.sync_copy(erel.at[pl.ds(base, EPW)], relv)

        def qloop(j, carry):
            r16 = relv[pl.ds(j * 16, 16)]
            qidxv[pl.ds(j * 16, 16)] = plsc.load_gather(qrelv, [r16])
            return carry

        lax.fori_loop(0, EPW // 16, qloop, 0)

        def chunk(k, carry):
            off = k * C
            cps = (
                pltpu.async_copy(rel_hbm.at[c2v.at[pl.ds(off, C)]], b_rel, sem),
                pltpu.async_copy(hid_hbm.at[subv.at[pl.ds(off, C)]], b_hs, sem),
                pltpu.async_copy(rel_hbm.at[qidxv.at[pl.ds(off, C)]], b_qr, sem),
                pltpu.async_copy(per_hbm.at[c6v.at[pl.ds(off, C)]], b_per, sem),
                pltpu.async_copy(pea_hbm.at[tabv.at[pl.ds(off, C)]], b_pea, sem),
            )
            for cp in cps:
                cp.wait()
            gb = base + off
            pltpu.sync_copy(b_rel, o_rel.at[pl.ds(gb, C)])
            pltpu.sync_copy(b_hs, o_hs.at[pl.ds(gb, C)])
            pltpu.sync_copy(b_qr, o_qr.at[pl.ds(gb, C)])
            pltpu.sync_copy(b_per, o_per.at[pl.ds(gb, C)])
            pltpu.sync_copy(b_pea, o_pea.at[pl.ds(gb, C)])
            return carry

        lax.fori_loop(0, NCHUNK, chunk, 0)

    return gather_k


def _dense_body(e_rel, e_hs, e_qr, e_per, e_pea,
                f1a, f1p, f1q, fw2, fb1, fb2,
                wg1, wg2, wg3, gb, wht1, wht2, htb,
                ws, wr, wqr, qrb, wal, o_msg):
    dot = functools.partial(jnp.dot, preferred_element_type=jnp.float32)
    rel = e_rel[...]
    hs = e_hs[...]
    qr = e_qr[...]
    h1 = _lrelu(dot(rel, f1a[...]) + dot(e_per[...], f1p[...])
                + dot(e_pea[...], f1q[...]) + fb1[...])
    fused = _lrelu(dot(h1, fw2[...]) + fb2[...])
    hr = fused + rel
    g = jax.nn.sigmoid(dot(hr, wg1[...]) + dot(qr, wg2[...])
                       + dot(hs, wg3[...]) + gb[...])
    u = g[:, :D]
    rst = g[:, D:]
    cand = jnp.tanh(dot(hr, wht1[...]) + dot(rst * hs, wht2[...]) + htb[...])
    msg = (1.0 - u) * hs + u * cand
    apre = _lrelu(dot(hs, ws[...]) + dot(hr, wr[...]) + dot(qr, wqr[...]) + qrb[...])
    alpha = jnp.sum(apre * wal[...], axis=1, keepdims=True)
    o_msg[...] = jax.nn.sigmoid(alpha) * msg


def _make_scatter(NE, NNP, EPW, NCHUNK):
    mesh = plsc.VectorSubcoreMesh(core_axis_name="c", subcore_axis_name="s")
    f32 = jnp.float32
    ROWS = NNP // NSUB

    @functools.partial(
        pl.kernel,
        out_type=jax.ShapeDtypeStruct((NCORE, NNP, D), f32),
        mesh=mesh,
        scratch_types=[
            pltpu.VMEM_SHARED((NNP, D), f32),
            pltpu.VMEM((NCHUNK, C), jnp.int32),
            pltpu.VMEM((C, D), f32),
        ],
    )
    def scat_k(msg_hbm, obj_hbm, zero_hbm, out_hbm, acc, objv, msgv):
        cid = lax.axis_index("c")
        sid = lax.axis_index("s")
        wid = sid * NCORE + cid
        pltpu.sync_copy(zero_hbm.at[pl.ds(sid * ROWS, ROWS)],
                        acc.at[pl.ds(sid * ROWS, ROWS)])
        pltpu.sync_copy(obj_hbm.at[wid], objv)
        plsc.subcore_barrier()
        base = wid * EPW

        def chunk(k, carry):
            pltpu.sync_copy(msg_hbm.at[pl.ds(base + k * C, C)], msgv)
            pltpu.sync_copy(msgv, acc.at[objv.at[k]], add=True)
            return carry

        lax.fori_loop(0, NCHUNK, chunk, 0)
        plsc.subcore_barrier()
        pltpu.sync_copy(acc.at[pl.ds(sid * ROWS, ROWS)],
                        out_hbm.at[cid, pl.ds(sid * ROWS, ROWS)])

    return scat_k


def _final_body(pp, wh, o):
    p = pp[0] + pp[1]
    o[...] = jnp.dot(p, wh[...], preferred_element_type=jnp.float32)


def kernel(q_sub, q_rel, hidden, edges, n_node, time_stamp, rel_table, pe_rel,
           pe_abs, gate_W_w, gate_W_b, gate_ht_w, gate_ht_b, fuse_w1, fuse_b1,
           fuse_w2, fuse_b2, Ws_w, Wr_w, Wqr_w, Wqr_b, walpha_w, Wh_w):
    f32 = jnp.float32
    NE = edges.shape[0]
    NN = hidden.shape[0]
    NR = rel_table.shape[0]
    NQ = q_rel.shape[0]
    EPW = NE // NW
    NCHUNK = EPW // C

    edges = edges.astype(jnp.int32)
    ec2 = edges[:, 2]
    ec6 = edges[:, 6]
    esub = edges[:, 4]
    erel = edges[:, 0]
    eobj = (edges[:, 5] % jnp.asarray(n_node, jnp.int32)).astype(jnp.int32)
    etab = (jnp.asarray(time_stamp, jnp.int32) - ec6).astype(jnp.int32)
    qrel_i = q_rel.astype(jnp.int32)

    gather_k = _make_gather(NE, NN, NR, NQ, EPW, NCHUNK)
    e_rel, e_hs, e_qr, e_per, e_pea = gather_k(
        rel_table, hidden, pe_rel, pe_abs, qrel_i, ec2, ec6, etab, esub, erel)

    BT = 1280
    grid = (NE // BT,)
    F1a = fuse_w1[:D]
    F1p = fuse_w1[D:D + DP]
    F1q = fuse_w1[D + DP:]
    Wg1 = gate_W_w[:D]
    Wg2 = gate_W_w[D:2 * D]
    Wg3 = gate_W_w[2 * D:]
    Wht1 = gate_ht_w[:D]
    Wht2 = gate_ht_w[D:]
    row = lambda v: v.reshape(1, -1)

    def eb(w):
        return pl.BlockSpec((BT, w), lambda i: (i, 0))

    def full(a):
        return pl.BlockSpec(a.shape, lambda i: (0,) * a.ndim)

    wargs = (F1a, F1p, F1q, fuse_w2, row(fuse_b1), row(fuse_b2),
             Wg1, Wg2, Wg3, row(gate_W_b), Wht1, Wht2, row(gate_ht_b),
             Ws_w, Wr_w, Wqr_w, row(Wqr_b), row(walpha_w[:, 0]))
    e_msg = pl.pallas_call(
        _dense_body,
        grid=grid,
        in_specs=[eb(D), eb(D), eb(D), eb(DP), eb(DP)] + [full(w) for w in wargs],
        out_specs=eb(D),
        out_shape=jax.ShapeDtypeStruct((NE, D), f32),
    )(e_rel, e_hs, e_qr, e_per, e_pea, *wargs)

    NNP = ((NN + 1279) // 1280) * 1280
    scat_k = _make_scatter(NE, NNP, EPW, NCHUNK)
    obj3 = eobj.reshape(NW, NCHUNK, C)
    zeros = jnp.zeros((NNP, D), f32)
    partials = scat_k(e_msg, obj3, zeros)

    BF = 1280
    out = pl.pallas_call(
        _final_body,
        grid=(NNP // BF,),
        in_specs=[pl.BlockSpec((NCORE, BF, D), lambda i: (0, i, 0)),
                  pl.BlockSpec((D, D), lambda i: (0, 0))],
        out_specs=pl.BlockSpec((BF, D), lambda i: (i, 0)),
        out_shape=jax.ShapeDtypeStruct((NNP, D), f32),
    )(partials, Wh_w)
    return out[:NN]


# speedup vs baseline: 4.1278x; 1.6005x over previous
import functools

import jax
import jax.numpy as jnp
from jax import lax
from jax.experimental import pallas as pl
from jax.experimental.pallas import tpu as pltpu
from jax.experimental.pallas import tpu_sc as plsc

D = 128
DA = 64
NCORE = 2
NSUB = 16
NW = NCORE * NSUB
C = 80


def _lrelu(t):
    return jnp.where(t >= 0, t, 0.01 * t)


def _sc_params():
    return pltpu.CompilerParams(needs_layout_passes=False,
                                use_tc_tiling_on_sc=False)


def _extract_col(ebuf, col, j):
    rows = lax.iota(jnp.int32, 16) + j * 16
    cols = jnp.full((16,), col, jnp.int32)
    return plsc.load_gather(ebuf, [rows, cols])


def _make_gather(NE, NQ, EPW, NCHUNK):
    mesh = plsc.VectorSubcoreMesh(core_axis_name="c", subcore_axis_name="s")
    f32 = jnp.float32

    @functools.partial(
        pl.kernel,
        out_type=tuple(jax.ShapeDtypeStruct((NE, D), f32) for _ in range(4)),
        mesh=mesh,
        scratch_types=[
            pltpu.VMEM((NQ,), jnp.int32),
            pltpu.VMEM((C, 7), jnp.int32),
            pltpu.VMEM((C,), jnp.int32),
            pltpu.VMEM((C,), jnp.int32),
            pltpu.VMEM((C,), jnp.int32),
            pltpu.VMEM((C,), jnp.int32),
            pltpu.VMEM((C, D), f32),
            pltpu.VMEM((C, D), f32),
            pltpu.VMEM((C, D), f32),
            pltpu.VMEM((C, D), f32),
            pltpu.SemaphoreType.DMA,
        ],
        compiler_params=_sc_params(),
    )
    def gather_k(rel_hbm, hid_hbm, p1_hbm, qrel_hbm, edges_hbm,
                 o_rel, o_hs, o_qr, o_p1,
                 qrelv, ebuf, c2v, c6v, subv, qidxv,
                 b_rel, b_hs, b_qr, b_p1, sem):
        wid = lax.axis_index("s") * NCORE + lax.axis_index("c")
        base = wid * EPW
        pltpu.sync_copy(qrel_hbm, qrelv)

        def chunk(k, carry):
            off = base + k * C
            pltpu.sync_copy(edges_hbm.at[pl.ds(off, C)], ebuf)
            for j in range(C // 16):
                sl = pl.ds(j * 16, 16)
                c2v[sl] = _extract_col(ebuf, 2, j)
                c6v[sl] = _extract_col(ebuf, 6, j)
                subv[sl] = _extract_col(ebuf, 4, j)
                qidxv[sl] = plsc.load_gather(qrelv, [_extract_col(ebuf, 0, j)])
            cps = (
                pltpu.async_copy(rel_hbm.at[c2v], b_rel, sem),
                pltpu.async_copy(hid_hbm.at[subv], b_hs, sem),
                pltpu.async_copy(rel_hbm.at[qidxv], b_qr, sem),
                pltpu.async_copy(p1_hbm.at[c6v], b_p1, sem),
            )
            for cp in cps:
                cp.wait()
            pltpu.sync_copy(b_rel, o_rel.at[pl.ds(off, C)])
            pltpu.sync_copy(b_hs, o_hs.at[pl.ds(off, C)])
            pltpu.sync_copy(b_qr, o_qr.at[pl.ds(off, C)])
            pltpu.sync_copy(b_p1, o_p1.at[pl.ds(off, C)])
            return carry

        lax.fori_loop(0, NCHUNK, chunk, 0)

    return gather_k


def _p1_body(pe, f1bc, fb1, o):
    o[...] = jnp.dot(pe[...], f1bc[...], preferred_element_type=jnp.float32) + fb1[...]


def _dense_body(e_rel, e_hs, e_qr, e_p1,
                f1a, fw2, fb2,
                wg1, wg2, wg3, gb, wht1, wht2, htb,
                ws, wr, wqr, qrb, wal, o_msg):
    dot = functools.partial(jnp.dot, preferred_element_type=jnp.float32)
    rel = e_rel[...]
    hs = e_hs[...]
    qr = e_qr[...]
    h1 = _lrelu(dot(rel, f1a[...]) + e_p1[...])
    fused = _lrelu(dot(h1, fw2[...]) + fb2[...])
    hr = fused + rel
    g = jax.nn.sigmoid(dot(hr, wg1[...]) + dot(qr, wg2[...])
                       + dot(hs, wg3[...]) + gb[...])
    u = g[:, :D]
    rst = g[:, D:]
    cand = jnp.tanh(dot(hr, wht1[...]) + dot(rst * hs, wht2[...]) + htb[...])
    msg = (1.0 - u) * hs + u * cand
    apre = _lrelu(dot(hs, ws[...]) + dot(hr, wr[...]) + dot(qr, wqr[...]) + qrb[...])
    alpha = jnp.sum(apre * wal[...], axis=1, keepdims=True)
    o_msg[...] = jax.nn.sigmoid(alpha) * msg


def _make_scatter(NE, NN, NNP, EPW, NCHUNK):
    mesh = plsc.VectorSubcoreMesh(core_axis_name="c", subcore_axis_name="s")
    f32 = jnp.float32
    ROWS = NNP // NSUB

    @functools.partial(
        pl.kernel,
        out_type=jax.ShapeDtypeStruct((NCORE, NNP, D), f32),
        mesh=mesh,
        scratch_types=[
            pltpu.VMEM_SHARED((NNP, D), f32),
            pltpu.VMEM((C, 7), jnp.int32),
            pltpu.VMEM((C,), jnp.int32),
            pltpu.VMEM((C, D), f32),
        ],
        compiler_params=_sc_params(),
    )
    def scat_k(msg_hbm, edges_hbm, zero_hbm, out_hbm, acc, ebuf, objv, msgv):
        cid = lax.axis_index("c")
        sid = lax.axis_index("s")
        wid = sid * NCORE + cid
        pltpu.sync_copy(zero_hbm.at[pl.ds(sid * ROWS, ROWS)],
                        acc.at[pl.ds(sid * ROWS, ROWS)])
        plsc.subcore_barrier()
        base = wid * EPW

        def chunk(k, carry):
            off = base + k * C
            pltpu.sync_copy(edges_hbm.at[pl.ds(off, C)], ebuf)
            pltpu.sync_copy(msg_hbm.at[pl.ds(off, C)], msgv)
            for j in range(C // 16):
                obj = _extract_col(ebuf, 5, j)
                objv[pl.ds(j * 16, 16)] = lax.rem(obj, jnp.full((16,), NN, jnp.int32))
            pltpu.sync_copy(msgv, acc.at[objv], add=True)
            return carry

        lax.fori_loop(0, NCHUNK, chunk, 0)
        plsc.subcore_barrier()
        pltpu.sync_copy(acc.at[pl.ds(sid * ROWS, ROWS)],
                        out_hbm.at[cid, pl.ds(sid * ROWS, ROWS)])

    return scat_k


def _final_body(pp, wh, o):
    p = pp[0] + pp[1]
    o[...] = jnp.dot(p, wh[...], preferred_element_type=jnp.float32)


def kernel(q_sub, q_rel, hidden, edges, n_node, time_stamp, rel_table, pe_rel,
           pe_abs, gate_W_w, gate_W_b, gate_ht_w, gate_ht_b, fuse_w1, fuse_b1,
           fuse_w2, fuse_b2, Ws_w, Wr_w, Wqr_w, Wqr_b, walpha_w, Wh_w):
    f32 = jnp.float32
    NE = edges.shape[0]
    NN = hidden.shape[0]
    NQ = q_rel.shape[0]
    NT = pe_rel.shape[0]
    DP = pe_rel.shape[1]
    EPW = NE // NW
    NCHUNK = EPW // C

    edges = edges.astype(jnp.int32)
    qrel_i = q_rel.astype(jnp.int32)

    pe_cat = jnp.concatenate(
        [pe_rel, pe_abs[jnp.asarray(time_stamp, jnp.int32) - jnp.arange(NT)]],
        axis=1)
    F1bc = fuse_w1[D:]
    BP = 2000
    p1 = pl.pallas_call(
        _p1_body,
        grid=(NT // BP,),
        in_specs=[pl.BlockSpec((BP, 2 * DP), lambda i: (i, 0)),
                  pl.BlockSpec((2 * DP, D), lambda i: (0, 0)),
                  pl.BlockSpec((1, D), lambda i: (0, 0))],
        out_specs=pl.BlockSpec((BP, D), lambda i: (i, 0)),
        out_shape=jax.ShapeDtypeStruct((NT, D), f32),
    )(pe_cat, F1bc, fuse_b1.reshape(1, D))

    gather_k = _make_gather(NE, NQ, EPW, NCHUNK)
    e_rel, e_hs, e_qr, e_p1 = gather_k(rel_table, hidden, p1, qrel_i, edges)

    BT = 1280
    grid = (NE // BT,)
    F1a = fuse_w1[:D]
    Wg1 = gate_W_w[:D]
    Wg2 = gate_W_w[D:2 * D]
    Wg3 = gate_W_w[2 * D:]
    Wht1 = gate_ht_w[:D]
    Wht2 = gate_ht_w[D:]
    row = lambda v: v.reshape(1, -1)

    def eb(w):
        return pl.BlockSpec((BT, w), lambda i: (i, 0))

    def full(a):
        return pl.BlockSpec(a.shape, lambda i: (0,) * a.ndim)

    wargs = (F1a, fuse_w2, row(fuse_b2),
             Wg1, Wg2, Wg3, row(gate_W_b), Wht1, Wht2, row(gate_ht_b),
             Ws_w, Wr_w, Wqr_w, row(Wqr_b), row(walpha_w[:, 0]))
    e_msg = pl.pallas_call(
        _dense_body,
        grid=grid,
        in_specs=[eb(D)] * 4 + [full(w) for w in wargs],
        out_specs=eb(D),
        out_shape=jax.ShapeDtypeStruct((NE, D), f32),
    )(e_rel, e_hs, e_qr, e_p1, *wargs)

    NNP = ((NN + 1279) // 1280) * 1280
    scat_k = _make_scatter(NE, NN, NNP, EPW, NCHUNK)
    zeros = jnp.zeros((NNP, D), f32)
    partials = scat_k(e_msg, edges, zeros)

    BF = 1280
    out = pl.pallas_call(
        _final_body,
        grid=(NNP // BF,),
        in_specs=[pl.BlockSpec((NCORE, BF, D), lambda i: (0, i, 0)),
                  pl.BlockSpec((D, D), lambda i: (0, 0))],
        out_specs=pl.BlockSpec((BF, D), lambda i: (i, 0)),
        out_shape=jax.ShapeDtypeStruct((NNP, D), f32),
    )(partials, Wh_w)
    return out[:NN]


# speedup vs baseline: 5.0828x; 1.2314x over previous
import functools

import jax
import jax.numpy as jnp
from jax import lax
from jax.experimental import pallas as pl
from jax.experimental.pallas import tpu as pltpu
from jax.experimental.pallas import tpu_sc as plsc

D = 128
DA = 64
NCORE = 2
NSUB = 16
NW = NCORE * NSUB
C = 80


def _lrelu(t):
    return jnp.where(t >= 0, t, 0.01 * t)


def _sc_params():
    return pltpu.CompilerParams(needs_layout_passes=False,
                                use_tc_tiling_on_sc=False)


def _extract16(ebuf, p, col, j):
    pp = jnp.full((16,), p, jnp.int32)
    rows = lax.iota(jnp.int32, 16) + j * 16
    cols = jnp.full((16,), col, jnp.int32)
    return plsc.load_gather(ebuf, [pp, rows, cols])


def _make_gather(NE, NQ, EPW, NCHUNK):
    mesh = plsc.VectorSubcoreMesh(core_axis_name="c", subcore_axis_name="s")
    f32 = jnp.float32

    @functools.partial(
        pl.kernel,
        out_type=tuple(jax.ShapeDtypeStruct((NE, D), f32) for _ in range(4)),
        mesh=mesh,
        scratch_types=[
            pltpu.VMEM((NQ,), jnp.int32),
            pltpu.VMEM((2, C, 8), jnp.int32),
            pltpu.VMEM((2, C), jnp.int32),
            pltpu.VMEM((2, C), jnp.int32),
            pltpu.VMEM((2, C), jnp.int32),
            pltpu.VMEM((2, C), jnp.int32),
            pltpu.VMEM((2, C, D), f32),
            pltpu.VMEM((2, C, D), f32),
            pltpu.VMEM((2, C, D), f32),
            pltpu.VMEM((2, C, D), f32),
            pltpu.SemaphoreType.DMA,
            pltpu.SemaphoreType.DMA,
            pltpu.SemaphoreType.DMA,
        ],
        compiler_params=_sc_params(),
    )
    def gather_k(rel_hbm, hid_hbm, p1_hbm, qrel_hbm, edges_hbm,
                 o_rel, o_hs, o_qr, o_p1,
                 qrelv, ebuf, c2v, c6v, subv, qidxv,
                 b_rel, b_hs, b_qr, b_p1, sem_e, sem_g, sem_w):
        wid = lax.axis_index("s") * NCORE + lax.axis_index("c")
        base = wid * EPW
        pltpu.sync_copy(qrel_hbm, qrelv)
        bufs = (b_rel, b_hs, b_qr, b_p1)
        tabs = (rel_hbm, hid_hbm, rel_hbm, p1_hbm)
        idxs = (c2v, subv, qidxv, c6v)
        outs = (o_rel, o_hs, o_qr, o_p1)

        pltpu.async_copy(edges_hbm.at[pl.ds(base, C)], ebuf.at[0], sem_e)

        def chunk(k, carry):
            p = lax.rem(k, 2)
            q = 1 - p
            off = base + k * C
            pltpu.make_async_copy(edges_hbm.at[pl.ds(off, C)],
                                  ebuf.at[p], sem_e).wait()

            @pl.when(k + 1 < NCHUNK)
            def _():
                pltpu.async_copy(edges_hbm.at[pl.ds(off + C, C)],
                                 ebuf.at[q], sem_e)

            for j in range(C // 16):
                sl = pl.ds(j * 16, 16)
                c2v[p, sl] = _extract16(ebuf, p, 2, j)
                c6v[p, sl] = _extract16(ebuf, p, 6, j)
                subv[p, sl] = _extract16(ebuf, p, 4, j)
                qidxv[p, sl] = plsc.load_gather(qrelv, [_extract16(ebuf, p, 0, j)])

            @pl.when(k >= 1)
            def _():
                for b in bufs:
                    pltpu.make_async_copy(o_rel.at[pl.ds(0, C)], b.at[q],
                                          sem_g).wait()

            @pl.when(k >= 2)
            def _():
                for b in bufs:
                    pltpu.make_async_copy(b.at[p], o_rel.at[pl.ds(0, C)],
                                          sem_w).wait()

            @pl.when(k >= 1)
            def _():
                for b, o in zip(bufs, outs):
                    pltpu.async_copy(b.at[q], o.at[pl.ds(off - C, C)], sem_w)

            for t, i, b in zip(tabs, idxs, bufs):
                pltpu.async_copy(t.at[i.at[p]], b.at[p], sem_g)
            return carry

        lax.fori_loop(0, NCHUNK, chunk, 0)
        pl_ = (NCHUNK - 1) % 2
        lastoff = base + (NCHUNK - 1) * C
        for b in bufs:
            pltpu.make_async_copy(o_rel.at[pl.ds(0, C)], b.at[pl_], sem_g).wait()
        for b, o in zip(bufs, outs):
            pltpu.async_copy(b.at[pl_], o.at[pl.ds(lastoff, C)], sem_w)
        for _ in range(2):
            for b in bufs:
                pltpu.make_async_copy(b.at[0], o_rel.at[pl.ds(0, C)], sem_w).wait()

    return gather_k


def _p1_body(pe, f1bc, fb1, o):
    o[...] = jnp.dot(pe[...], f1bc[...], preferred_element_type=jnp.float32) + fb1[...]


def _dense_body(e_rel, e_hs, e_qr, e_p1,
                f1a, fw2, fb2,
                wg1, wg2, wg3, gb, wht1, wht2, htb,
                ws, wr, wqr, qrb, wal, o_msg):
    dot = functools.partial(jnp.dot, preferred_element_type=jnp.float32)
    rel = e_rel[...]
    hs = e_hs[...]
    qr = e_qr[...]
    h1 = _lrelu(dot(rel, f1a[...]) + e_p1[...])
    fused = _lrelu(dot(h1, fw2[...]) + fb2[...])
    hr = fused + rel
    g = jax.nn.sigmoid(dot(hr, wg1[...]) + dot(qr, wg2[...])
                       + dot(hs, wg3[...]) + gb[...])
    u = g[:, :D]
    rst = g[:, D:]
    cand = jnp.tanh(dot(hr, wht1[...]) + dot(rst * hs, wht2[...]) + htb[...])
    msg = (1.0 - u) * hs + u * cand
    apre = _lrelu(dot(hs, ws[...]) + dot(hr, wr[...]) + dot(qr, wqr[...]) + qrb[...])
    alpha = jnp.sum(apre * wal[...], axis=1, keepdims=True)
    o_msg[...] = jax.nn.sigmoid(alpha) * msg


def _make_scatter(NE, NN, NNP, EPW, NCHUNK):
    mesh = plsc.VectorSubcoreMesh(core_axis_name="c", subcore_axis_name="s")
    f32 = jnp.float32
    ROWS = NNP // NSUB

    @functools.partial(
        pl.kernel,
        out_type=jax.ShapeDtypeStruct((NCORE, NNP, D), f32),
        mesh=mesh,
        scratch_types=[
            pltpu.VMEM_SHARED((NNP, D), f32),
            pltpu.VMEM((2, C, 8), jnp.int32),
            pltpu.VMEM((2, C), jnp.int32),
            pltpu.VMEM((2, C, D), f32),
            pltpu.SemaphoreType.DMA,
            pltpu.SemaphoreType.DMA,
        ],
        compiler_params=_sc_params(),
    )
    def scat_k(msg_hbm, edges_hbm, zero_hbm, out_hbm, acc, ebuf, objv, msgv,
               sem_e, sem_m):
        cid = lax.axis_index("c")
        sid = lax.axis_index("s")
        wid = sid * NCORE + cid
        base = wid * EPW
        pltpu.async_copy(edges_hbm.at[pl.ds(base, C)], ebuf.at[0], sem_e)
        pltpu.async_copy(msg_hbm.at[pl.ds(base, C)], msgv.at[0], sem_m)
        pltpu.sync_copy(zero_hbm.at[pl.ds(sid * ROWS, ROWS)],
                        acc.at[pl.ds(sid * ROWS, ROWS)])
        plsc.subcore_barrier()

        def chunk(k, carry):
            p = lax.rem(k, 2)
            q = 1 - p
            off = base + k * C
            pltpu.make_async_copy(edges_hbm.at[pl.ds(off, C)],
                                  ebuf.at[p], sem_e).wait()
            pltpu.make_async_copy(msg_hbm.at[pl.ds(off, C)],
                                  msgv.at[p], sem_m).wait()

            @pl.when(k + 1 < NCHUNK)
            def _():
                pltpu.async_copy(edges_hbm.at[pl.ds(off + C, C)],
                                 ebuf.at[q], sem_e)
                pltpu.async_copy(msg_hbm.at[pl.ds(off + C, C)],
                                 msgv.at[q], sem_m)

            nn16 = jnp.full((16,), NN, jnp.int32)
            for j in range(C // 16):
                objv[p, pl.ds(j * 16, 16)] = lax.rem(_extract16(ebuf, p, 5, j),
                                                     nn16)
            pltpu.sync_copy(msgv.at[p], acc.at[objv.at[p]], add=True)
            return carry

        lax.fori_loop(0, NCHUNK, chunk, 0)
        plsc.subcore_barrier()
        pltpu.sync_copy(acc.at[pl.ds(sid * ROWS, ROWS)],
                        out_hbm.at[cid, pl.ds(sid * ROWS, ROWS)])

    return scat_k


def _final_body(pp, wh, o):
    p = pp[0] + pp[1]
    o[...] = jnp.dot(p, wh[...], preferred_element_type=jnp.float32)


def kernel(q_sub, q_rel, hidden, edges, n_node, time_stamp, rel_table, pe_rel,
           pe_abs, gate_W_w, gate_W_b, gate_ht_w, gate_ht_b, fuse_w1, fuse_b1,
           fuse_w2, fuse_b2, Ws_w, Wr_w, Wqr_w, Wqr_b, walpha_w, Wh_w):
    f32 = jnp.float32
    NE = edges.shape[0]
    NN = hidden.shape[0]
    NQ = q_rel.shape[0]
    NT = pe_rel.shape[0]
    DP = pe_rel.shape[1]
    EPW = NE // NW
    NCHUNK = EPW // C

    edges8 = jnp.pad(edges.astype(jnp.int32), ((0, 0), (0, 1)))
    qrel_i = q_rel.astype(jnp.int32)

    pe_cat = jnp.concatenate(
        [pe_rel, pe_abs[jnp.asarray(time_stamp, jnp.int32) - jnp.arange(NT)]],
        axis=1)
    F1bc = fuse_w1[D:]
    BP = 2000
    p1 = pl.pallas_call(
        _p1_body,
        grid=(NT // BP,),
        in_specs=[pl.BlockSpec((BP, 2 * DP), lambda i: (i, 0)),
                  pl.BlockSpec((2 * DP, D), lambda i: (0, 0)),
                  pl.BlockSpec((1, D), lambda i: (0, 0))],
        out_specs=pl.BlockSpec((BP, D), lambda i: (i, 0)),
        out_shape=jax.ShapeDtypeStruct((NT, D), f32),
    )(pe_cat, F1bc, fuse_b1.reshape(1, D))

    gather_k = _make_gather(NE, NQ, EPW, NCHUNK)
    e_rel, e_hs, e_qr, e_p1 = gather_k(rel_table, hidden, p1, qrel_i, edges8)

    BT = 1280
    grid = (NE // BT,)
    F1a = fuse_w1[:D]
    Wg1 = gate_W_w[:D]
    Wg2 = gate_W_w[D:2 * D]
    Wg3 = gate_W_w[2 * D:]
    Wht1 = gate_ht_w[:D]
    Wht2 = gate_ht_w[D:]
    row = lambda v: v.reshape(1, -1)

    def eb(w):
        return pl.BlockSpec((BT, w), lambda i: (i, 0))

    def full(a):
        return pl.BlockSpec(a.shape, lambda i: (0,) * a.ndim)

    wargs = (F1a, fuse_w2, row(fuse_b2),
             Wg1, Wg2, Wg3, row(gate_W_b), Wht1, Wht2, row(gate_ht_b),
             Ws_w, Wr_w, Wqr_w, row(Wqr_b), row(walpha_w[:, 0]))
    e_msg = pl.pallas_call(
        _dense_body,
        grid=grid,
        in_specs=[eb(D)] * 4 + [full(w) for w in wargs],
        out_specs=eb(D),
        out_shape=jax.ShapeDtypeStruct((NE, D), f32),
    )(e_rel, e_hs, e_qr, e_p1, *wargs)

    NNP = ((NN + 1279) // 1280) * 1280
    scat_k = _make_scatter(NE, NN, NNP, EPW, NCHUNK)
    zeros = jnp.zeros((NNP, D), f32)
    partials = scat_k(e_msg, edges8, zeros)

    BF = 1280
    out = pl.pallas_call(
        _final_body,
        grid=(NNP // BF,),
        in_specs=[pl.BlockSpec((NCORE, BF, D), lambda i: (0, i, 0)),
                  pl.BlockSpec((D, D), lambda i: (0, 0))],
        out_specs=pl.BlockSpec((BF, D), lambda i: (i, 0)),
        out_shape=jax.ShapeDtypeStruct((NNP, D), f32),
    )(partials, Wh_w)
    return out[:NN]
